# baseline (speedup 1.0000x reference)
import functools

import jax
import jax.numpy as jnp
from jax import lax
from jax.experimental import pallas as pl
from jax.experimental.pallas import tpu as pltpu
from jax.experimental.pallas import tpu_sc as plsc

N_NODES = 10000
N_EDGES = 320000
FEAT = 128

NC = 2
NS = 16
CHUNK = 128
N_CHUNKS = N_EDGES // CHUNK
CHUNKS_PER_CORE = N_CHUNKS // NC
BASE_CHUNKS = CHUNKS_PER_CORE // NS
ITERS = BASE_CHUNKS + 1
IDX_LEN = ITERS * CHUNK
N_PAD = 10240
ROWS_PER_SUB = N_PAD // NS

ROW_BLK = 1000
GRID = N_NODES // ROW_BLK


def _linear_scale(x, W, b, d0, d1):
    def body(x_ref, w_ref, b_ref, d0_ref, d1_ref, g_ref, nd_ref):
        h = lax.dot_general(
            x_ref[...], w_ref[...], (((1,), (1,)), ((), ())),
            preferred_element_type=jnp.float32) + b_ref[...]
        deg = d0_ref[...] + d1_ref[...]
        nd = jnp.where(deg > 0, lax.rsqrt(deg), jnp.zeros_like(deg))
        nd_ref[...] = nd
        g_ref[...] = h * nd

    return pl.pallas_call(
        body,
        grid=(GRID,),
        in_specs=[pl.BlockSpec((ROW_BLK, FEAT), lambda i: (i, 0)),
                  pl.BlockSpec((FEAT, FEAT), lambda i: (0, 0)),
                  pl.BlockSpec((1, FEAT), lambda i: (0, 0)),
                  pl.BlockSpec((ROW_BLK, 1), lambda i: (i, 0)),
                  pl.BlockSpec((ROW_BLK, 1), lambda i: (i, 0))],
        out_specs=[pl.BlockSpec((ROW_BLK, FEAT), lambda i: (i, 0)),
                   pl.BlockSpec((ROW_BLK, 1), lambda i: (i, 0))],
        out_shape=[jax.ShapeDtypeStruct((N_NODES, FEAT), jnp.float32),
                   jax.ShapeDtypeStruct((N_NODES, 1), jnp.float32)],
    )(x, W, b.reshape(1, FEAT), d0, d1)


def _worker_range(core, sid):
    start = core * CHUNKS_PER_CORE + sid * BASE_CHUNKS + jnp.minimum(sid, NC)
    return start


def _fill(ref, offset, length, value):
    vec = jnp.full((16,), value, ref.dtype)

    @pl.loop(0, length // 16)
    def _(t):
        ref[pl.ds(offset + t * 16, 16)] = vec


def _degree(r):
    mesh = plsc.VectorSubcoreMesh(core_axis_name="c", subcore_axis_name="s")

    @functools.partial(
        pl.kernel,
        out_type=jax.ShapeDtypeStruct((NC, N_PAD), jnp.float32),
        mesh=mesh,
        scratch_types=[
            pltpu.VMEM_SHARED((N_PAD,), jnp.float32),
            pltpu.VMEM((IDX_LEN,), jnp.int32),
            pltpu.VMEM((CHUNK,), jnp.float32),
            pltpu.VMEM((ROWS_PER_SUB,), jnp.float32),
        ],
    )
    def k(r_hbm, deg_hbm, deg_sh, ridx_v, ones_v, zbuf_v):
        core = lax.axis_index("c")
        sid = lax.axis_index("s")
        start = _worker_range(core, sid)

        _fill(zbuf_v, 0, ROWS_PER_SUB, 0.0)
        _fill(ones_v, 0, CHUNK, 1.0)
        sl = pl.ds(sid * ROWS_PER_SUB, ROWS_PER_SUB)
        pltpu.sync_copy(zbuf_v, deg_sh.at[sl])

        pltpu.sync_copy(r_hbm.at[pl.ds(start * CHUNK, BASE_CHUNKS * CHUNK)],
                        ridx_v.at[pl.ds(0, BASE_CHUNKS * CHUNK)])

        @pl.when(sid < NC)
        def _():
            pltpu.sync_copy(
                r_hbm.at[pl.ds((start + BASE_CHUNKS) * CHUNK, CHUNK)],
                ridx_v.at[pl.ds(BASE_CHUNKS * CHUNK, CHUNK)])

        @pl.when(sid >= NC)
        def _():
            _fill(ridx_v, BASE_CHUNKS * CHUNK, CHUNK, N_NODES + sid)

        plsc.subcore_barrier()

        @pl.loop(0, ITERS)
        def _(j):
            pltpu.sync_copy(ones_v, deg_sh.at[ridx_v.at[pl.ds(j * CHUNK, CHUNK)]],
                            add=True)

        plsc.subcore_barrier()
        pltpu.sync_copy(deg_sh.at[sl], deg_hbm.at[core, sl])

    return k(r)


def _aggregate(g, edge_index):
    mesh = plsc.VectorSubcoreMesh(core_axis_name="c", subcore_axis_name="s")

    @functools.partial(
        pl.kernel,
        out_type=jax.ShapeDtypeStruct((NC, N_PAD, FEAT), jnp.float32),
        mesh=mesh,
        scratch_types=[
            pltpu.VMEM_SHARED((N_PAD, FEAT), jnp.float32),
            pltpu.VMEM((2, CHUNK), jnp.int32),
            pltpu.VMEM((2, CHUNK), jnp.int32),
            pltpu.VMEM((CHUNK,), jnp.int32),
            pltpu.VMEM((CHUNK,), jnp.int32),
            pltpu.VMEM((CHUNK, FEAT), jnp.float32),
            pltpu.VMEM((CHUNK, FEAT), jnp.float32),
            pltpu.SemaphoreType.DMA,
            pltpu.SemaphoreType.DMA,
            pltpu.SemaphoreType.DMA,
            pltpu.SemaphoreType.DMA,
            pltpu.SemaphoreType.DMA,
            pltpu.SemaphoreType.DMA,
        ],
    )
    def k(g_hbm, ei_hbm, out_hbm, acc_sh, eidx_a, eidx_b, ridx_a, ridx_b,
          rows_a, rows_b, semg_a, semg_b, semi_a, semi_b, sems_a, sems_b):
        core = lax.axis_index("c")
        sid = lax.axis_index("s")
        start = core * CHUNKS_PER_CORE + sid * BASE_CHUNKS

        @pl.loop(0, CHUNK)
        def _(i):
            @pl.loop(0, FEAT // 16)
            def _(t):
                rows_a[i, pl.ds(t * 16, 16)] = jnp.zeros((16,), jnp.float32)

        @pl.loop(0, ROWS_PER_SUB // CHUNK)
        def _(z):
            pltpu.sync_copy(
                rows_a, acc_sh.at[pl.ds(sid * ROWS_PER_SUB + z * CHUNK, CHUNK)])

        plsc.subcore_barrier()

        def idx_start(j, eidx, sem):
            pltpu.async_copy(
                ei_hbm.at[:, pl.ds((start + j) * CHUNK, CHUNK)], eidx, sem)

        def idx_wait(eidx, sem):
            pltpu.make_async_copy(ei_hbm.at[:, pl.ds(0, CHUNK)], eidx, sem).wait()

        def gather_start(eidx, rows, sem):
            pltpu.async_copy(g_hbm.at[eidx.at[1]], rows, sem)

        def gather_wait(eidx, rows, sem):
            pltpu.make_async_copy(g_hbm.at[eidx.at[1]], rows, sem).wait()

        def ridx_copy(eidx, ridx):
            @pl.loop(0, CHUNK // 16)
            def _(t):
                ridx[pl.ds(t * 16, 16)] = eidx[0, pl.ds(t * 16, 16)]

        def scatter_start(rows, ridx, sem):
            pltpu.async_copy(rows, acc_sh.at[ridx], sem, add=True)

        def scatter_wait(rows, ridx, sem):
            pltpu.make_async_copy(rows, acc_sh.at[ridx], sem).wait()

        pltpu.sync_copy(ei_hbm.at[:, pl.ds(start * CHUNK, CHUNK)], eidx_a)
        gather_start(eidx_a, rows_a, semg_a)
        idx_start(1, eidx_b, semi_b)
        idx_wait(eidx_b, semi_b)
        gather_start(eidx_b, rows_b, semg_b)

        @pl.loop(0, BASE_CHUNKS // 2)
        def _(kk):
            j0 = 2 * kk
            more = j0 + 2 < BASE_CHUNKS
            gather_wait(eidx_a, rows_a, semg_a)
            ridx_copy(eidx_a, ridx_a)
            scatter_start(rows_a, ridx_a, sems_a)

            @pl.when(more)
            def _():
                idx_start(j0 + 2, eidx_a, semi_a)

            gather_wait(eidx_b, rows_b, semg_b)
            ridx_copy(eidx_b, ridx_b)
            scatter_start(rows_b, ridx_b, sems_b)

            @pl.when(more)
            def _():
                idx_start(j0 + 3, eidx_b, semi_b)
                idx_wait(eidx_a, semi_a)
                scatter_wait(rows_a, ridx_a, sems_a)
                gather_start(eidx_a, rows_a, semg_a)
                idx_wait(eidx_b, semi_b)
                scatter_wait(rows_b, ridx_b, sems_b)
                gather_start(eidx_b, rows_b, semg_b)

        scatter_wait(rows_a, ridx_a, sems_a)
        scatter_wait(rows_b, ridx_b, sems_b)

        @pl.when(sid < NC)
        def _():
            tail = core * CHUNKS_PER_CORE + NS * BASE_CHUNKS + sid
            pltpu.sync_copy(ei_hbm.at[:, pl.ds(tail * CHUNK, CHUNK)], eidx_a)
            gather_start(eidx_a, rows_a, semg_a)
            gather_wait(eidx_a, rows_a, semg_a)
            pltpu.sync_copy(rows_a, acc_sh.at[eidx_a.at[0]], add=True)

        plsc.subcore_barrier()
        rsl = pl.ds(sid * ROWS_PER_SUB, ROWS_PER_SUB)
        pltpu.sync_copy(acc_sh.at[rsl], out_hbm.at[core, rsl])

    return k(g, edge_index)


def _combine(q0, q1, nd):
    def body(q0_ref, q1_ref, nd_ref, o_ref):
        o_ref[...] = (q0_ref[...] + q1_ref[...]) * nd_ref[...]

    return pl.pallas_call(
        body,
        grid=(GRID,),
        in_specs=[pl.BlockSpec((ROW_BLK, FEAT), lambda i: (i, 0)),
                  pl.BlockSpec((ROW_BLK, FEAT), lambda i: (i, 0)),
                  pl.BlockSpec((ROW_BLK, 1), lambda i: (i, 0))],
        out_specs=pl.BlockSpec((ROW_BLK, FEAT), lambda i: (i, 0)),
        out_shape=jax.ShapeDtypeStruct((N_NODES, FEAT), jnp.float32),
    )(q0, q1, nd)


def kernel(x, edge_index, W, b):
    r = edge_index[0]
    degp = _degree(r)
    d0 = degp[0, :N_NODES].reshape(N_NODES, 1)
    d1 = degp[1, :N_NODES].reshape(N_NODES, 1)
    g, nd = _linear_scale(x, W, b, d0, d1)
    outp = _aggregate(g, edge_index)
    return _combine(outp[0, :N_NODES], outp[1, :N_NODES], nd)


# speedup vs baseline: 1.0973x; 1.0973x over previous
import functools

import jax
import jax.numpy as jnp
from jax import lax
from jax.experimental import pallas as pl
from jax.experimental.pallas import tpu as pltpu
from jax.experimental.pallas import tpu_sc as plsc

N_NODES = 10000
N_EDGES = 320000
FEAT = 128

NC = 2
NS = 16
CHUNK = 128
N_CHUNKS = N_EDGES // CHUNK
CHUNKS_PER_CORE = N_CHUNKS // NC
BASE_CHUNKS = CHUNKS_PER_CORE // NS
ITERS = BASE_CHUNKS + 1
IDX_LEN = ITERS * CHUNK
N_PAD = 10112
ROWS_PER_SUB = N_PAD // NS
DEG_PAD = 10240
DEG_PER_SUB = DEG_PAD // NS

ROW_BLK = 1000
GRID = N_NODES // ROW_BLK


def _linear_scale(x, W, b, d0, d1):
    def body(x_ref, w_ref, b_ref, d0_ref, d1_ref, g_ref, nd_ref):
        h = lax.dot_general(
            x_ref[...], w_ref[...], (((1,), (1,)), ((), ())),
            preferred_element_type=jnp.float32) + b_ref[...]
        deg = d0_ref[...] + d1_ref[...]
        nd = jnp.where(deg > 0, lax.rsqrt(deg), jnp.zeros_like(deg))
        nd_ref[...] = nd
        g_ref[...] = h * nd

    return pl.pallas_call(
        body,
        grid=(GRID,),
        in_specs=[pl.BlockSpec((ROW_BLK, FEAT), lambda i: (i, 0)),
                  pl.BlockSpec((FEAT, FEAT), lambda i: (0, 0)),
                  pl.BlockSpec((1, FEAT), lambda i: (0, 0)),
                  pl.BlockSpec((ROW_BLK, 1), lambda i: (i, 0)),
                  pl.BlockSpec((ROW_BLK, 1), lambda i: (i, 0))],
        out_specs=[pl.BlockSpec((ROW_BLK, FEAT), lambda i: (i, 0)),
                   pl.BlockSpec((ROW_BLK, 1), lambda i: (i, 0))],
        out_shape=[jax.ShapeDtypeStruct((N_NODES, FEAT), jnp.float32),
                   jax.ShapeDtypeStruct((N_NODES, 1), jnp.float32)],
    )(x, W, b.reshape(1, FEAT), d0, d1)


def _worker_range(core, sid):
    start = core * CHUNKS_PER_CORE + sid * BASE_CHUNKS + jnp.minimum(sid, NC)
    return start


def _fill(ref, offset, length, value):
    vec = jnp.full((16,), value, ref.dtype)

    @pl.loop(0, length // 16)
    def _(t):
        ref[pl.ds(offset + t * 16, 16)] = vec


def _degree(r):
    mesh = plsc.VectorSubcoreMesh(core_axis_name="c", subcore_axis_name="s")

    @functools.partial(
        pl.kernel,
        out_type=jax.ShapeDtypeStruct((NC, DEG_PAD), jnp.float32),
        mesh=mesh,
        scratch_types=[
            pltpu.VMEM_SHARED((DEG_PAD,), jnp.float32),
            pltpu.VMEM((IDX_LEN,), jnp.int32),
            pltpu.VMEM((CHUNK,), jnp.float32),
            pltpu.VMEM((640,), jnp.float32),
        ],
    )
    def k(r_hbm, deg_hbm, deg_sh, ridx_v, ones_v, zbuf_v):
        core = lax.axis_index("c")
        sid = lax.axis_index("s")
        start = _worker_range(core, sid)

        _fill(zbuf_v, 0, DEG_PER_SUB, 0.0)
        _fill(ones_v, 0, CHUNK, 1.0)
        sl = pl.ds(sid * DEG_PER_SUB, DEG_PER_SUB)
        pltpu.sync_copy(zbuf_v, deg_sh.at[sl])

        pltpu.sync_copy(r_hbm.at[pl.ds(start * CHUNK, BASE_CHUNKS * CHUNK)],
                        ridx_v.at[pl.ds(0, BASE_CHUNKS * CHUNK)])

        @pl.when(sid < NC)
        def _():
            pltpu.sync_copy(
                r_hbm.at[pl.ds((start + BASE_CHUNKS) * CHUNK, CHUNK)],
                ridx_v.at[pl.ds(BASE_CHUNKS * CHUNK, CHUNK)])

        @pl.when(sid >= NC)
        def _():
            _fill(ridx_v, BASE_CHUNKS * CHUNK, CHUNK, N_NODES + sid)

        plsc.subcore_barrier()

        @pl.loop(0, ITERS)
        def _(j):
            pltpu.sync_copy(ones_v, deg_sh.at[ridx_v.at[pl.ds(j * CHUNK, CHUNK)]],
                            add=True)

        plsc.subcore_barrier()
        pltpu.sync_copy(deg_sh.at[sl], deg_hbm.at[core, sl])

    return k(r)


NSLOT = 3
RING_ITERS = BASE_CHUNKS // NSLOT


def _aggregate(g, r, c):
    mesh = plsc.VectorSubcoreMesh(core_axis_name="c", subcore_axis_name="s")

    @functools.partial(
        pl.kernel,
        out_type=jax.ShapeDtypeStruct((NC, N_PAD, FEAT), jnp.float32),
        mesh=mesh,
        scratch_types=[
            pltpu.VMEM_SHARED((N_PAD, FEAT), jnp.float32),
            [pltpu.VMEM((CHUNK,), jnp.int32)] * NSLOT,
            [pltpu.VMEM((CHUNK,), jnp.int32)] * NSLOT,
            [pltpu.VMEM((CHUNK, FEAT), jnp.float32)] * NSLOT,
            [pltpu.SemaphoreType.DMA] * NSLOT,
            [pltpu.SemaphoreType.DMA] * NSLOT,
            [pltpu.SemaphoreType.DMA] * NSLOT,
            [pltpu.SemaphoreType.DMA] * NSLOT,
        ],
    )
    def k(g_hbm, r_hbm, c_hbm, out_hbm, acc_sh, cidx, ridx, rows,
          semg, sems, semc, semr):
        core = lax.axis_index("c")
        sid = lax.axis_index("s")
        start = core * CHUNKS_PER_CORE + sid * BASE_CHUNKS

        @pl.loop(0, CHUNK)
        def _(i):
            @pl.loop(0, FEAT // 16)
            def _(t):
                rows[0][i, pl.ds(t * 16, 16)] = jnp.zeros((16,), jnp.float32)

        @pl.loop(0, ROWS_PER_SUB // CHUNK)
        def _(z):
            pltpu.sync_copy(
                rows[0],
                acc_sh.at[pl.ds(sid * ROWS_PER_SUB + z * CHUNK, CHUNK)])

        pltpu.sync_copy(
            rows[0].at[pl.ds(0, ROWS_PER_SUB % CHUNK)],
            acc_sh.at[pl.ds(sid * ROWS_PER_SUB + ROWS_PER_SUB - ROWS_PER_SUB % CHUNK,
                            ROWS_PER_SUB % CHUNK)])

        plsc.subcore_barrier()

        def cidx_start(j, x):
            pltpu.async_copy(c_hbm.at[pl.ds((start + j) * CHUNK, CHUNK)],
                             cidx[x], semc[x])

        def cidx_wait(x):
            pltpu.make_async_copy(c_hbm.at[pl.ds(0, CHUNK)],
                                  cidx[x], semc[x]).wait()

        def ridx_start(j, x):
            pltpu.async_copy(r_hbm.at[pl.ds((start + j) * CHUNK, CHUNK)],
                             ridx[x], semr[x])

        def ridx_wait(x):
            pltpu.make_async_copy(r_hbm.at[pl.ds(0, CHUNK)],
                                  ridx[x], semr[x]).wait()

        def gather_start(x):
            pltpu.async_copy(g_hbm.at[cidx[x]], rows[x], semg[x])

        def gather_wait(x):
            pltpu.make_async_copy(g_hbm.at[cidx[x]], rows[x], semg[x]).wait()

        def scatter_start(x):
            pltpu.async_copy(rows[x], acc_sh.at[ridx[x]], sems[x], add=True)

        def scatter_wait(x):
            pltpu.make_async_copy(rows[x], acc_sh.at[ridx[x]], sems[x]).wait()

        for x in range(NSLOT):
            cidx_start(x, x)
            ridx_start(x, x)
        for x in range(NSLOT):
            cidx_wait(x)
            gather_start(x)

        @pl.loop(0, RING_ITERS)
        def _(m):
            more = m + 1 < RING_ITERS
            for x in range(NSLOT):
                gather_wait(x)
                ridx_wait(x)
                scatter_start(x)

                @pl.when(more)
                def _():
                    cidx_start(NSLOT * (m + 1) + x, x)

            @pl.when(more)
            def _():
                for x in range(NSLOT):
                    scatter_wait(x)
                    cidx_wait(x)
                    gather_start(x)
                    ridx_start(NSLOT * (m + 1) + x, x)

        for x in range(NSLOT):
            scatter_wait(x)

        @pl.when(sid < NC)
        def _():
            tail = core * CHUNKS_PER_CORE + NS * BASE_CHUNKS + sid
            pltpu.sync_copy(c_hbm.at[pl.ds(tail * CHUNK, CHUNK)], cidx[0])
            pltpu.sync_copy(r_hbm.at[pl.ds(tail * CHUNK, CHUNK)], ridx[0])
            gather_start(0)
            gather_wait(0)
            pltpu.sync_copy(rows[0], acc_sh.at[ridx[0]], add=True)

        plsc.subcore_barrier()
        rsl = pl.ds(sid * ROWS_PER_SUB, ROWS_PER_SUB)
        pltpu.sync_copy(acc_sh.at[rsl], out_hbm.at[core, rsl])

    return k(g, r, c)


def _combine(q0, q1, nd):
    def body(q0_ref, q1_ref, nd_ref, o_ref):
        o_ref[...] = (q0_ref[...] + q1_ref[...]) * nd_ref[...]

    return pl.pallas_call(
        body,
        grid=(GRID,),
        in_specs=[pl.BlockSpec((ROW_BLK, FEAT), lambda i: (i, 0)),
                  pl.BlockSpec((ROW_BLK, FEAT), lambda i: (i, 0)),
                  pl.BlockSpec((ROW_BLK, 1), lambda i: (i, 0))],
        out_specs=pl.BlockSpec((ROW_BLK, FEAT), lambda i: (i, 0)),
        out_shape=jax.ShapeDtypeStruct((N_NODES, FEAT), jnp.float32),
    )(q0, q1, nd)


def kernel(x, edge_index, W, b):
    r = edge_index[0]
    degp = _degree(r)
    d0 = degp[0, :N_NODES].reshape(N_NODES, 1)
    d1 = degp[1, :N_NODES].reshape(N_NODES, 1)
    g, nd = _linear_scale(x, W, b, d0, d1)
    outp = _aggregate(g, r, edge_index[1])
    return _combine(outp[0, :N_NODES], outp[1, :N_NODES], nd)


# speedup vs baseline: 1.1484x; 1.0465x over previous
import functools

import jax
import jax.numpy as jnp
from jax import lax
from jax.experimental import pallas as pl
from jax.experimental.pallas import tpu as pltpu
from jax.experimental.pallas import tpu_sc as plsc

N_NODES = 10000
N_EDGES = 320000
FEAT = 128

NC = 2
NS = 16
CHUNK = 128
N_CHUNKS = N_EDGES // CHUNK
CHUNKS_PER_CORE = N_CHUNKS // NC
BASE_CHUNKS = CHUNKS_PER_CORE // NS
ITERS = BASE_CHUNKS + 1
IDX_LEN = ITERS * CHUNK
N_PAD = 10112
ROWS_PER_SUB = N_PAD // NS
DEG_PAD = 10240
DEG_PER_SUB = DEG_PAD // NS

ROW_BLK = 1000
GRID = N_NODES // ROW_BLK


def _linear_scale(x, W, b, degp):
    def body(x_ref, w_ref, b_ref, dp_ref, g_ref, nd_ref):
        h = lax.dot_general(
            x_ref[...], w_ref[...], (((1,), (1,)), ((), ())),
            preferred_element_type=jnp.float32) + b_ref[...]
        deg = dp_ref[0] + dp_ref[1]
        nd = jnp.where(deg > 0, lax.rsqrt(deg), jnp.zeros_like(deg))
        nd_ref[...] = nd
        g_ref[...] = h * nd

    return pl.pallas_call(
        body,
        grid=(GRID,),
        in_specs=[pl.BlockSpec((ROW_BLK, FEAT), lambda i: (i, 0)),
                  pl.BlockSpec((FEAT, FEAT), lambda i: (0, 0)),
                  pl.BlockSpec((1, FEAT), lambda i: (0, 0)),
                  pl.BlockSpec((NC, ROW_BLK, 1), lambda i: (0, i, 0))],
        out_specs=[pl.BlockSpec((ROW_BLK, FEAT), lambda i: (i, 0)),
                   pl.BlockSpec((ROW_BLK, 1), lambda i: (i, 0))],
        out_shape=[jax.ShapeDtypeStruct((N_NODES, FEAT), jnp.float32),
                   jax.ShapeDtypeStruct((N_NODES, 1), jnp.float32)],
    )(x, W, b.reshape(1, FEAT), degp.reshape(NC, DEG_PAD, 1))


def _worker_range(core, sid):
    start = core * CHUNKS_PER_CORE + sid * BASE_CHUNKS + jnp.minimum(sid, NC)
    return start


def _fill(ref, offset, length, value):
    vec = jnp.full((16,), value, ref.dtype)

    @pl.loop(0, length // 16)
    def _(t):
        ref[pl.ds(offset + t * 16, 16)] = vec


def _degree(r):
    mesh = plsc.VectorSubcoreMesh(core_axis_name="c", subcore_axis_name="s")

    @functools.partial(
        pl.kernel,
        out_type=jax.ShapeDtypeStruct((NC, DEG_PAD), jnp.float32),
        mesh=mesh,
        scratch_types=[
            pltpu.VMEM_SHARED((DEG_PAD,), jnp.float32),
            pltpu.VMEM((IDX_LEN,), jnp.int32),
            pltpu.VMEM((CHUNK,), jnp.float32),
            pltpu.VMEM((640,), jnp.float32),
        ],
    )
    def k(r_hbm, deg_hbm, deg_sh, ridx_v, ones_v, zbuf_v):
        core = lax.axis_index("c")
        sid = lax.axis_index("s")
        start = _worker_range(core, sid)

        _fill(zbuf_v, 0, DEG_PER_SUB, 0.0)
        _fill(ones_v, 0, CHUNK, 1.0)
        sl = pl.ds(sid * DEG_PER_SUB, DEG_PER_SUB)
        pltpu.sync_copy(zbuf_v, deg_sh.at[sl])

        pltpu.sync_copy(r_hbm.at[pl.ds(start * CHUNK, BASE_CHUNKS * CHUNK)],
                        ridx_v.at[pl.ds(0, BASE_CHUNKS * CHUNK)])

        @pl.when(sid < NC)
        def _():
            pltpu.sync_copy(
                r_hbm.at[pl.ds((start + BASE_CHUNKS) * CHUNK, CHUNK)],
                ridx_v.at[pl.ds(BASE_CHUNKS * CHUNK, CHUNK)])

        @pl.when(sid >= NC)
        def _():
            _fill(ridx_v, BASE_CHUNKS * CHUNK, CHUNK, N_NODES + sid)

        plsc.subcore_barrier()

        @pl.loop(0, ITERS)
        def _(j):
            pltpu.sync_copy(ones_v, deg_sh.at[ridx_v.at[pl.ds(j * CHUNK, CHUNK)]],
                            add=True)

        plsc.subcore_barrier()
        pltpu.sync_copy(deg_sh.at[sl], deg_hbm.at[core, sl])

    return k(r)


NSLOT = 3
RING_ITERS = BASE_CHUNKS // NSLOT


def _aggregate(g, r, c):
    mesh = plsc.VectorSubcoreMesh(core_axis_name="c", subcore_axis_name="s")

    @functools.partial(
        pl.kernel,
        out_type=jax.ShapeDtypeStruct((NC, N_PAD, FEAT), jnp.float32),
        mesh=mesh,
        scratch_types=[
            pltpu.VMEM_SHARED((N_PAD, FEAT), jnp.float32),
            [pltpu.VMEM((CHUNK,), jnp.int32)] * NSLOT,
            [pltpu.VMEM((CHUNK,), jnp.int32)] * NSLOT,
            [pltpu.VMEM((CHUNK, FEAT), jnp.float32)] * NSLOT,
            [pltpu.SemaphoreType.DMA] * NSLOT,
            [pltpu.SemaphoreType.DMA] * NSLOT,
            [pltpu.SemaphoreType.DMA] * NSLOT,
            [pltpu.SemaphoreType.DMA] * NSLOT,
        ],
    )
    def k(g_hbm, r_hbm, c_hbm, out_hbm, acc_sh, cidx, ridx, rows,
          semg, sems, semc, semr):
        core = lax.axis_index("c")
        sid = lax.axis_index("s")
        start = core * CHUNKS_PER_CORE + sid * BASE_CHUNKS

        @pl.loop(0, CHUNK)
        def _(i):
            @pl.loop(0, FEAT // 16)
            def _(t):
                rows[0][i, pl.ds(t * 16, 16)] = jnp.zeros((16,), jnp.float32)

        @pl.loop(0, ROWS_PER_SUB // CHUNK)
        def _(z):
            pltpu.sync_copy(
                rows[0],
                acc_sh.at[pl.ds(sid * ROWS_PER_SUB + z * CHUNK, CHUNK)])

        pltpu.sync_copy(
            rows[0].at[pl.ds(0, ROWS_PER_SUB % CHUNK)],
            acc_sh.at[pl.ds(sid * ROWS_PER_SUB + ROWS_PER_SUB - ROWS_PER_SUB % CHUNK,
                            ROWS_PER_SUB % CHUNK)])

        plsc.subcore_barrier()

        def cidx_start(j, x):
            pltpu.async_copy(c_hbm.at[pl.ds((start + j) * CHUNK, CHUNK)],
                             cidx[x], semc[x])

        def cidx_wait(x):
            pltpu.make_async_copy(c_hbm.at[pl.ds(0, CHUNK)],
                                  cidx[x], semc[x]).wait()

        def ridx_start(j, x):
            pltpu.async_copy(r_hbm.at[pl.ds((start + j) * CHUNK, CHUNK)],
                             ridx[x], semr[x])

        def ridx_wait(x):
            pltpu.make_async_copy(r_hbm.at[pl.ds(0, CHUNK)],
                                  ridx[x], semr[x]).wait()

        def gather_start(x):
            pltpu.async_copy(g_hbm.at[cidx[x]], rows[x], semg[x])

        def gather_wait(x):
            pltpu.make_async_copy(g_hbm.at[cidx[x]], rows[x], semg[x]).wait()

        def scatter_start(x):
            pltpu.async_copy(rows[x], acc_sh.at[ridx[x]], sems[x], add=True)

        def scatter_wait(x):
            pltpu.make_async_copy(rows[x], acc_sh.at[ridx[x]], sems[x]).wait()

        for x in range(NSLOT):
            cidx_start(x, x)
            ridx_start(x, x)
        for x in range(NSLOT):
            cidx_wait(x)
            gather_start(x)

        @pl.loop(0, RING_ITERS)
        def _(m):
            more = m + 1 < RING_ITERS
            for x in range(NSLOT):
                gather_wait(x)
                ridx_wait(x)
                scatter_start(x)

                @pl.when(more)
                def _():
                    cidx_start(NSLOT * (m + 1) + x, x)

            @pl.when(more)
            def _():
                for x in range(NSLOT):
                    scatter_wait(x)
                    cidx_wait(x)
                    gather_start(x)
                    ridx_start(NSLOT * (m + 1) + x, x)

        for x in range(NSLOT):
            scatter_wait(x)

        @pl.when(sid < NC)
        def _():
            tail = core * CHUNKS_PER_CORE + NS * BASE_CHUNKS + sid
            pltpu.sync_copy(c_hbm.at[pl.ds(tail * CHUNK, CHUNK)], cidx[0])
            pltpu.sync_copy(r_hbm.at[pl.ds(tail * CHUNK, CHUNK)], ridx[0])
            gather_start(0)
            gather_wait(0)
            pltpu.sync_copy(rows[0], acc_sh.at[ridx[0]], add=True)

        plsc.subcore_barrier()
        rsl = pl.ds(sid * ROWS_PER_SUB, ROWS_PER_SUB)
        pltpu.sync_copy(acc_sh.at[rsl], out_hbm.at[core, rsl])

    return k(g, r, c)


def _combine(outp, nd):
    def body(qp_ref, nd_ref, o_ref):
        o_ref[...] = (qp_ref[0] + qp_ref[1]) * nd_ref[...]

    return pl.pallas_call(
        body,
        grid=(GRID,),
        in_specs=[pl.BlockSpec((NC, ROW_BLK, FEAT), lambda i: (0, i, 0)),
                  pl.BlockSpec((ROW_BLK, 1), lambda i: (i, 0))],
        out_specs=pl.BlockSpec((ROW_BLK, FEAT), lambda i: (i, 0)),
        out_shape=jax.ShapeDtypeStruct((N_NODES, FEAT), jnp.float32),
    )(outp, nd)


def kernel(x, edge_index, W, b):
    r = edge_index[0]
    degp = _degree(r)
    g, nd = _linear_scale(x, W, b, degp)
    outp = _aggregate(g, r, edge_index[1])
    return _combine(outp, nd)


# speedup vs baseline: 1.1860x; 1.0328x over previous
import functools

import jax
import jax.numpy as jnp
from jax import lax
from jax.experimental import pallas as pl
from jax.experimental.pallas import tpu as pltpu
from jax.experimental.pallas import tpu_sc as plsc

N_NODES = 10000
N_EDGES = 320000
FEAT = 128

NC = 2
NS = 16
CHUNK = 128
N_CHUNKS = N_EDGES // CHUNK
CHUNKS_PER_CORE = N_CHUNKS // NC
BASE_CHUNKS = CHUNKS_PER_CORE // NS
ITERS = BASE_CHUNKS + 1
IDX_LEN = ITERS * CHUNK
N_PAD = 10112
ROWS_PER_SUB = N_PAD // NS
DEG_PAD = 10240
DEG_PER_SUB = DEG_PAD // NS

ROW_BLK = 1000
GRID = N_NODES // ROW_BLK


def _linear_scale(x, W, b, degp):
    def body(x_ref, w_ref, b_ref, dp_ref, g_ref, nd_ref):
        h = lax.dot_general(
            x_ref[...], w_ref[...], (((1,), (1,)), ((), ())),
            preferred_element_type=jnp.float32) + b_ref[...]
        deg = dp_ref[0] + dp_ref[1]
        nd = jnp.where(deg > 0, lax.rsqrt(deg), jnp.zeros_like(deg))
        nd_ref[...] = nd
        g_ref[...] = h * nd

    return pl.pallas_call(
        body,
        grid=(GRID,),
        in_specs=[pl.BlockSpec((ROW_BLK, FEAT), lambda i: (i, 0)),
                  pl.BlockSpec((FEAT, FEAT), lambda i: (0, 0)),
                  pl.BlockSpec((1, FEAT), lambda i: (0, 0)),
                  pl.BlockSpec((NC, ROW_BLK, 1), lambda i: (0, i, 0))],
        out_specs=[pl.BlockSpec((ROW_BLK, FEAT), lambda i: (i, 0)),
                   pl.BlockSpec((ROW_BLK, 1), lambda i: (i, 0))],
        out_shape=[jax.ShapeDtypeStruct((N_NODES, FEAT), jnp.float32),
                   jax.ShapeDtypeStruct((N_NODES, 1), jnp.float32)],
    )(x, W, b.reshape(1, FEAT), degp.reshape(NC, DEG_PAD, 1))


def _worker_range(core, sid):
    start = core * CHUNKS_PER_CORE + sid * BASE_CHUNKS + jnp.minimum(sid, NC)
    return start


def _fill(ref, offset, length, value):
    vec = jnp.full((16,), value, ref.dtype)

    @pl.loop(0, length // 16)
    def _(t):
        ref[pl.ds(offset + t * 16, 16)] = vec


def _degree(r):
    mesh = plsc.VectorSubcoreMesh(core_axis_name="c", subcore_axis_name="s")

    @functools.partial(
        pl.kernel,
        out_type=jax.ShapeDtypeStruct((NC, DEG_PAD), jnp.float32),
        mesh=mesh,
        scratch_types=[
            pltpu.VMEM_SHARED((DEG_PAD,), jnp.float32),
            pltpu.VMEM((IDX_LEN,), jnp.int32),
            pltpu.VMEM((CHUNK,), jnp.float32),
            pltpu.VMEM((640,), jnp.float32),
            pltpu.SemaphoreType.DMA,
            pltpu.SemaphoreType.DMA,
        ],
    )
    def k(r_hbm, deg_hbm, deg_sh, ridx_v, ones_v, zbuf_v, semi, sem):
        core = lax.axis_index("c")
        sid = lax.axis_index("s")
        start = _worker_range(core, sid)

        pltpu.async_copy(r_hbm.at[pl.ds(start * CHUNK, BASE_CHUNKS * CHUNK)],
                         ridx_v.at[pl.ds(0, BASE_CHUNKS * CHUNK)], semi)

        _fill(zbuf_v, 0, DEG_PER_SUB, 0.0)
        _fill(ones_v, 0, CHUNK, 1.0)
        sl = pl.ds(sid * DEG_PER_SUB, DEG_PER_SUB)
        pltpu.sync_copy(zbuf_v, deg_sh.at[sl])

        @pl.when(sid < NC)
        def _():
            pltpu.sync_copy(
                r_hbm.at[pl.ds((start + BASE_CHUNKS) * CHUNK, CHUNK)],
                ridx_v.at[pl.ds(BASE_CHUNKS * CHUNK, CHUNK)])

        @pl.when(sid >= NC)
        def _():
            _fill(ridx_v, BASE_CHUNKS * CHUNK, CHUNK, N_NODES + sid)

        pltpu.make_async_copy(
            r_hbm.at[pl.ds(0, BASE_CHUNKS * CHUNK)],
            ridx_v.at[pl.ds(0, BASE_CHUNKS * CHUNK)], semi).wait()
        plsc.subcore_barrier()

        @pl.loop(0, ITERS)
        def _(j):
            pltpu.async_copy(ones_v, deg_sh.at[ridx_v.at[pl.ds(j * CHUNK, CHUNK)]],
                             sem, add=True)

        @pl.loop(0, ITERS)
        def _(j):
            pltpu.make_async_copy(
                ones_v, deg_sh.at[ridx_v.at[pl.ds(0, CHUNK)]], sem).wait()

        plsc.subcore_barrier()
        pltpu.sync_copy(deg_sh.at[sl], deg_hbm.at[core, sl])

    return k(r)


NSLOT = 3
RING_ITERS = BASE_CHUNKS // NSLOT


def _aggregate(g, r, c):
    mesh = plsc.VectorSubcoreMesh(core_axis_name="c", subcore_axis_name="s")

    @functools.partial(
        pl.kernel,
        out_type=jax.ShapeDtypeStruct((NC, N_PAD, FEAT), jnp.float32),
        mesh=mesh,
        scratch_types=[
            pltpu.VMEM_SHARED((N_PAD, FEAT), jnp.float32),
            [pltpu.VMEM((CHUNK,), jnp.int32)] * NSLOT,
            [pltpu.VMEM((CHUNK,), jnp.int32)] * NSLOT,
            [pltpu.VMEM((CHUNK, FEAT), jnp.float32)] * NSLOT,
            [pltpu.SemaphoreType.DMA] * NSLOT,
            [pltpu.SemaphoreType.DMA] * NSLOT,
            [pltpu.SemaphoreType.DMA] * NSLOT,
            [pltpu.SemaphoreType.DMA] * NSLOT,
        ],
    )
    def k(g_hbm, r_hbm, c_hbm, out_hbm, acc_sh, cidx, ridx, rows,
          semg, sems, semc, semr):
        core = lax.axis_index("c")
        sid = lax.axis_index("s")
        start = core * CHUNKS_PER_CORE + sid * BASE_CHUNKS

        @pl.loop(0, CHUNK)
        def _(i):
            @pl.loop(0, FEAT // 16)
            def _(t):
                rows[0][i, pl.ds(t * 16, 16)] = jnp.zeros((16,), jnp.float32)

        @pl.loop(0, ROWS_PER_SUB // CHUNK)
        def _(z):
            pltpu.sync_copy(
                rows[0],
                acc_sh.at[pl.ds(sid * ROWS_PER_SUB + z * CHUNK, CHUNK)])

        pltpu.sync_copy(
            rows[0].at[pl.ds(0, ROWS_PER_SUB % CHUNK)],
            acc_sh.at[pl.ds(sid * ROWS_PER_SUB + ROWS_PER_SUB - ROWS_PER_SUB % CHUNK,
                            ROWS_PER_SUB % CHUNK)])

        plsc.subcore_barrier()

        def cidx_start(j, x):
            pltpu.async_copy(c_hbm.at[pl.ds((start + j) * CHUNK, CHUNK)],
                             cidx[x], semc[x])

        def cidx_wait(x):
            pltpu.make_async_copy(c_hbm.at[pl.ds(0, CHUNK)],
                                  cidx[x], semc[x]).wait()

        def ridx_start(j, x):
            pltpu.async_copy(r_hbm.at[pl.ds((start + j) * CHUNK, CHUNK)],
                             ridx[x], semr[x])

        def ridx_wait(x):
            pltpu.make_async_copy(r_hbm.at[pl.ds(0, CHUNK)],
                                  ridx[x], semr[x]).wait()

        def gather_start(x):
            pltpu.async_copy(g_hbm.at[cidx[x]], rows[x], semg[x])

        def gather_wait(x):
            pltpu.make_async_copy(g_hbm.at[cidx[x]], rows[x], semg[x]).wait()

        def scatter_start(x):
            pltpu.async_copy(rows[x], acc_sh.at[ridx[x]], sems[x], add=True)

        def scatter_wait(x):
            pltpu.make_async_copy(rows[x], acc_sh.at[ridx[x]], sems[x]).wait()

        for x in range(NSLOT):
            cidx_start(x, x)
            ridx_start(x, x)
        for x in range(NSLOT):
            cidx_wait(x)
            gather_start(x)

        @pl.loop(0, RING_ITERS)
        def _(m):
            more = m + 1 < RING_ITERS
            for x in range(NSLOT):
                gather_wait(x)
                ridx_wait(x)
                scatter_start(x)

                @pl.when(more)
                def _():
                    cidx_start(NSLOT * (m + 1) + x, x)

            @pl.when(more)
            def _():
                for x in range(NSLOT):
                    scatter_wait(x)
                    cidx_wait(x)
                    gather_start(x)
                    ridx_start(NSLOT * (m + 1) + x, x)

        for x in range(NSLOT):
            scatter_wait(x)

        @pl.when(sid < NC)
        def _():
            tail = core * CHUNKS_PER_CORE + NS * BASE_CHUNKS + sid
            pltpu.sync_copy(c_hbm.at[pl.ds(tail * CHUNK, CHUNK)], cidx[0])
            pltpu.sync_copy(r_hbm.at[pl.ds(tail * CHUNK, CHUNK)], ridx[0])
            gather_start(0)
            gather_wait(0)
            pltpu.sync_copy(rows[0], acc_sh.at[ridx[0]], add=True)

        plsc.subcore_barrier()
        rsl = pl.ds(sid * ROWS_PER_SUB, ROWS_PER_SUB)
        pltpu.sync_copy(acc_sh.at[rsl], out_hbm.at[core, rsl])

    return k(g, r, c)


def _combine(outp, nd):
    def body(qp_ref, nd_ref, o_ref):
        o_ref[...] = (qp_ref[0] + qp_ref[1]) * nd_ref[...]

    return pl.pallas_call(
        body,
        grid=(GRID,),
        in_specs=[pl.BlockSpec((NC, ROW_BLK, FEAT), lambda i: (0, i, 0)),
                  pl.BlockSpec((ROW_BLK, 1), lambda i: (i, 0))],
        out_specs=pl.BlockSpec((ROW_BLK, FEAT), lambda i: (i, 0)),
        out_shape=jax.ShapeDtypeStruct((N_NODES, FEAT), jnp.float32),
    )(outp, nd)


def kernel(x, edge_index, W, b):
    r = edge_index[0]
    degp = _degree(r)
    g, nd = _linear_scale(x, W, b, degp)
    outp = _aggregate(g, r, edge_index[1])
    return _combine(outp, nd)


# speedup vs baseline: 1.1869x; 1.0007x over previous
import functools

import jax
import jax.numpy as jnp
from jax import lax
from jax.experimental import pallas as pl
from jax.experimental.pallas import tpu as pltpu
from jax.experimental.pallas import tpu_sc as plsc

N_NODES = 10000
N_EDGES = 320000
FEAT = 128

NC = 2
NS = 16
CHUNK = 128
N_CHUNKS = N_EDGES // CHUNK
CHUNKS_PER_CORE = N_CHUNKS // NC
BASE_CHUNKS = CHUNKS_PER_CORE // NS
ITERS = BASE_CHUNKS + 1
IDX_LEN = ITERS * CHUNK
N_PAD = 10112
ROWS_PER_SUB = N_PAD // NS
DEG_PAD = 10240
DEG_PER_SUB = DEG_PAD // NS

ROW_BLK = 1000
GRID = N_NODES // ROW_BLK


def _linear_scale(x, W, b, degp):
    def body(x_ref, w_ref, b_ref, dp_ref, g_ref, nd_ref):
        h = lax.dot_general(
            x_ref[...], w_ref[...], (((1,), (1,)), ((), ())),
            preferred_element_type=jnp.float32) + b_ref[...]
        deg = dp_ref[0] + dp_ref[1]
        nd = jnp.where(deg > 0, lax.rsqrt(deg), jnp.zeros_like(deg))
        nd_ref[...] = nd
        g_ref[...] = h * nd

    return pl.pallas_call(
        body,
        grid=(GRID,),
        in_specs=[pl.BlockSpec((ROW_BLK, FEAT), lambda i: (i, 0)),
                  pl.BlockSpec((FEAT, FEAT), lambda i: (0, 0)),
                  pl.BlockSpec((1, FEAT), lambda i: (0, 0)),
                  pl.BlockSpec((NC, ROW_BLK, 1), lambda i: (0, i, 0))],
        out_specs=[pl.BlockSpec((ROW_BLK, FEAT), lambda i: (i, 0)),
                   pl.BlockSpec((ROW_BLK, 1), lambda i: (i, 0))],
        out_shape=[jax.ShapeDtypeStruct((N_NODES, FEAT), jnp.float32),
                   jax.ShapeDtypeStruct((N_NODES, 1), jnp.float32)],
    )(x, W, b.reshape(1, FEAT), degp.reshape(NC, DEG_PAD, 1))


def _worker_range(core, sid):
    start = core * CHUNKS_PER_CORE + sid * BASE_CHUNKS + jnp.minimum(sid, NC)
    return start


def _fill(ref, offset, length, value):
    vec = jnp.full((16,), value, ref.dtype)

    @pl.loop(0, length // 16)
    def _(t):
        ref[pl.ds(offset + t * 16, 16)] = vec


def _degree(r):
    mesh = plsc.VectorSubcoreMesh(core_axis_name="c", subcore_axis_name="s")

    @functools.partial(
        pl.kernel,
        out_type=jax.ShapeDtypeStruct((NC, DEG_PAD), jnp.float32),
        mesh=mesh,
        scratch_types=[
            pltpu.VMEM_SHARED((DEG_PAD,), jnp.float32),
            pltpu.VMEM((IDX_LEN,), jnp.int32),
            pltpu.VMEM((CHUNK,), jnp.float32),
            pltpu.VMEM((640,), jnp.float32),
            pltpu.SemaphoreType.DMA,
            pltpu.SemaphoreType.DMA,
        ],
    )
    def k(r_hbm, deg_hbm, deg_sh, ridx_v, ones_v, zbuf_v, semi, sem):
        core = lax.axis_index("c")
        sid = lax.axis_index("s")
        start = _worker_range(core, sid)

        pltpu.async_copy(r_hbm.at[pl.ds(start * CHUNK, BASE_CHUNKS * CHUNK)],
                         ridx_v.at[pl.ds(0, BASE_CHUNKS * CHUNK)], semi)

        _fill(zbuf_v, 0, DEG_PER_SUB, 0.0)
        _fill(ones_v, 0, CHUNK, 1.0)
        sl = pl.ds(sid * DEG_PER_SUB, DEG_PER_SUB)
        pltpu.sync_copy(zbuf_v, deg_sh.at[sl])

        @pl.when(sid < NC)
        def _():
            pltpu.sync_copy(
                r_hbm.at[pl.ds((start + BASE_CHUNKS) * CHUNK, CHUNK)],
                ridx_v.at[pl.ds(BASE_CHUNKS * CHUNK, CHUNK)])

        @pl.when(sid >= NC)
        def _():
            _fill(ridx_v, BASE_CHUNKS * CHUNK, CHUNK, N_NODES + sid)

        pltpu.make_async_copy(
            r_hbm.at[pl.ds(0, BASE_CHUNKS * CHUNK)],
            ridx_v.at[pl.ds(0, BASE_CHUNKS * CHUNK)], semi).wait()
        plsc.subcore_barrier()

        @pl.loop(0, ITERS)
        def _(j):
            pltpu.async_copy(ones_v, deg_sh.at[ridx_v.at[pl.ds(j * CHUNK, CHUNK)]],
                             sem, add=True)

        @pl.loop(0, ITERS)
        def _(j):
            pltpu.make_async_copy(
                ones_v, deg_sh.at[ridx_v.at[pl.ds(0, CHUNK)]], sem).wait()

        plsc.subcore_barrier()
        pltpu.sync_copy(deg_sh.at[sl], deg_hbm.at[core, sl])

    return k(r)


NSLOT = 3
RING_ITERS = BASE_CHUNKS // NSLOT


def _aggregate(g, r, c):
    mesh = plsc.VectorSubcoreMesh(core_axis_name="c", subcore_axis_name="s")

    @functools.partial(
        pl.kernel,
        out_type=jax.ShapeDtypeStruct((NC, N_PAD, FEAT), jnp.float32),
        mesh=mesh,
        scratch_types=[
            pltpu.VMEM_SHARED((N_PAD, FEAT), jnp.float32),
            [pltpu.VMEM((CHUNK,), jnp.int32)] * NSLOT,
            [pltpu.VMEM((CHUNK,), jnp.int32)] * NSLOT,
            [pltpu.VMEM((CHUNK, FEAT), jnp.float32)] * NSLOT,
            [pltpu.SemaphoreType.DMA] * NSLOT,
            [pltpu.SemaphoreType.DMA] * NSLOT,
            [pltpu.SemaphoreType.DMA] * NSLOT,
            [pltpu.SemaphoreType.DMA] * NSLOT,
        ],
    )
    def k(g_hbm, r_hbm, c_hbm, out_hbm, acc_sh, cidx, ridx, rows,
          semg, sems, semc, semr):
        core = lax.axis_index("c")
        sid = lax.axis_index("s")
        start = core * CHUNKS_PER_CORE + sid * BASE_CHUNKS

        @pl.loop(0, CHUNK)
        def _(i):
            @pl.loop(0, FEAT // 16)
            def _(t):
                rows[0][i, pl.ds(t * 16, 16)] = jnp.zeros((16,), jnp.float32)

        @pl.loop(0, ROWS_PER_SUB // CHUNK)
        def _(z):
            pltpu.async_copy(
                rows[0],
                acc_sh.at[pl.ds(sid * ROWS_PER_SUB + z * CHUNK, CHUNK)],
                semg[0])

        pltpu.async_copy(
            rows[0].at[pl.ds(0, ROWS_PER_SUB % CHUNK)],
            acc_sh.at[pl.ds(sid * ROWS_PER_SUB + ROWS_PER_SUB - ROWS_PER_SUB % CHUNK,
                            ROWS_PER_SUB % CHUNK)],
            semg[0])

        @pl.loop(0, ROWS_PER_SUB // CHUNK)
        def _(z):
            pltpu.make_async_copy(
                rows[0],
                acc_sh.at[pl.ds(sid * ROWS_PER_SUB, CHUNK)], semg[0]).wait()

        pltpu.make_async_copy(
            rows[0].at[pl.ds(0, ROWS_PER_SUB % CHUNK)],
            acc_sh.at[pl.ds(sid * ROWS_PER_SUB, ROWS_PER_SUB % CHUNK)],
            semg[0]).wait()

        plsc.subcore_barrier()

        def cidx_start(j, x):
            pltpu.async_copy(c_hbm.at[pl.ds((start + j) * CHUNK, CHUNK)],
                             cidx[x], semc[x])

        def cidx_wait(x):
            pltpu.make_async_copy(c_hbm.at[pl.ds(0, CHUNK)],
                                  cidx[x], semc[x]).wait()

        def ridx_start(j, x):
            pltpu.async_copy(r_hbm.at[pl.ds((start + j) * CHUNK, CHUNK)],
                             ridx[x], semr[x])

        def ridx_wait(x):
            pltpu.make_async_copy(r_hbm.at[pl.ds(0, CHUNK)],
                                  ridx[x], semr[x]).wait()

        def gather_start(x):
            pltpu.async_copy(g_hbm.at[cidx[x]], rows[x], semg[x])

        def gather_wait(x):
            pltpu.make_async_copy(g_hbm.at[cidx[x]], rows[x], semg[x]).wait()

        def scatter_start(x):
            pltpu.async_copy(rows[x], acc_sh.at[ridx[x]], sems[x], add=True)

        def scatter_wait(x):
            pltpu.make_async_copy(rows[x], acc_sh.at[ridx[x]], sems[x]).wait()

        for x in range(NSLOT):
            cidx_start(x, x)
            ridx_start(x, x)
        for x in range(NSLOT):
            cidx_wait(x)
            gather_start(x)

        @pl.loop(0, RING_ITERS)
        def _(m):
            more = m + 1 < RING_ITERS
            for x in range(NSLOT):
                gather_wait(x)
                ridx_wait(x)
                scatter_start(x)

                @pl.when(more)
                def _():
                    cidx_start(NSLOT * (m + 1) + x, x)

            @pl.when(more)
            def _():
                for x in range(NSLOT):
                    scatter_wait(x)
                    cidx_wait(x)
                    gather_start(x)
                    ridx_start(NSLOT * (m + 1) + x, x)

        for x in range(NSLOT):
            scatter_wait(x)

        @pl.when(sid < NC)
        def _():
            tail = core * CHUNKS_PER_CORE + NS * BASE_CHUNKS + sid
            pltpu.sync_copy(c_hbm.at[pl.ds(tail * CHUNK, CHUNK)], cidx[0])
            pltpu.sync_copy(r_hbm.at[pl.ds(tail * CHUNK, CHUNK)], ridx[0])
            gather_start(0)
            gather_wait(0)
            pltpu.sync_copy(rows[0], acc_sh.at[ridx[0]], add=True)

        plsc.subcore_barrier()
        rsl = pl.ds(sid * ROWS_PER_SUB, ROWS_PER_SUB)
        pltpu.sync_copy(acc_sh.at[rsl], out_hbm.at[core, rsl])

    return k(g, r, c)


def _combine(outp, nd):
    def body(qp_ref, nd_ref, o_ref):
        o_ref[...] = (qp_ref[0] + qp_ref[1]) * nd_ref[...]

    return pl.pallas_call(
        body,
        grid=(GRID,),
        in_specs=[pl.BlockSpec((NC, ROW_BLK, FEAT), lambda i: (0, i, 0)),
                  pl.BlockSpec((ROW_BLK, 1), lambda i: (i, 0))],
        out_specs=pl.BlockSpec((ROW_BLK, FEAT), lambda i: (i, 0)),
        out_shape=jax.ShapeDtypeStruct((N_NODES, FEAT), jnp.float32),
    )(outp, nd)


def kernel(x, edge_index, W, b):
    r = edge_index[0]
    degp = _degree(r)
    g, nd = _linear_scale(x, W, b, degp)
    outp = _aggregate(g, r, edge_index[1])
    return _combine(outp, nd)
